# Initial kernel scaffold; baseline (speedup 1.0000x reference)
import jax
import jax.numpy as jnp
from jax.experimental import pallas as pl


def kernel(features, centres, centre_labels, weight):
    raise NotImplementedError("write your pallas kernel here")


# speedup vs baseline: 1.3885x; 1.3885x over previous
import functools

import jax
import jax.numpy as jnp
from jax.experimental import pallas as pl

NUM_CLASSES = 1000
NUM_NEIGHBOURS = 200
SIGMA = 10.0
GC = 1.0 / (2.0 * SIGMA ** 2)

_BR = 256
_BC = 2048


def _d2_kernel(q_ref, c_ref, o_ref):
    q = q_ref[...]
    c = c_ref[...]
    q2 = jnp.sum(q * q, axis=1, keepdims=True)
    c2 = jnp.sum(c * c, axis=1)[None, :]
    dot = jax.lax.dot_general(
        q, c, (((1,), (1,)), ((), ())),
        preferred_element_type=jnp.float32,
        precision=jax.lax.Precision.DEFAULT)
    o_ref[...] = q2 + c2 - 2.0 * dot


def _pairwise_d2(features, centres_pad, n_pad):
    b, d = features.shape
    grid = (b // _BR, n_pad // _BC)
    return pl.pallas_call(
        _d2_kernel,
        grid=grid,
        in_specs=[
            pl.BlockSpec((_BR, d), lambda i, j: (i, 0)),
            pl.BlockSpec((_BC, d), lambda i, j: (j, 0)),
        ],
        out_specs=pl.BlockSpec((_BR, _BC), lambda i, j: (i, j)),
        out_shape=jax.ShapeDtypeStruct((b, n_pad), jnp.float32),
    )(features, centres_pad)


def kernel(features, centres, centre_labels, weight):
    b, d = features.shape
    n = centres.shape[0]
    n_pad = ((n + _BC - 1) // _BC) * _BC
    centres_pad = jnp.pad(centres, ((0, n_pad - n), (0, 0)),
                          constant_values=1e3)
    d2 = _pairwise_d2(features, centres_pad, n_pad)

    neg_vals, idx = jax.lax.top_k(-d2, NUM_NEIGHBOURS)
    d2k = -neg_vals
    kw = jnp.exp(weight)
    contrib = jnp.exp(-d2k * GC) * kw[idx]
    labels = centre_labels[idx]
    p = jnp.zeros((b, NUM_CLASSES), dtype=jnp.float32)
    p = p.at[jnp.arange(b)[:, None], labels].add(contrib)
    p = jnp.where(p == 0.0, 1e-10, p)
    p = p / jnp.sum(p, axis=1, keepdims=True)
    return (jnp.log(p), p)


# speedup vs baseline: 13.3835x; 9.6390x over previous
import jax
import jax.numpy as jnp
from jax.experimental import pallas as pl

NUM_CLASSES = 1000
NN = 200
SIGMA = 10.0
GC = 1.0 / (2.0 * SIGMA ** 2)

_BR = 256
_BC = 2048
_J = 16
_PASSES = 5


def _d2_minmax_kernel(q_ref, c_ref, d2_ref, mn_ref, mx_ref, *, n):
    j = pl.program_id(1)
    q = q_ref[...]
    c = c_ref[...]
    q2 = jnp.sum(q * q, axis=1, keepdims=True)
    c2 = jnp.sum(c * c, axis=1)[None, :]
    dot = jax.lax.dot_general(q, c, (((1,), (1,)), ((), ())),
                              preferred_element_type=jnp.float32)
    d2 = q2 + c2 - 2.0 * dot
    d2_ref[...] = d2
    col = j * d2.shape[1] + jax.lax.broadcasted_iota(jnp.int32, d2.shape, 1)
    valid = col < n
    tmn = jnp.broadcast_to(jnp.min(d2, axis=1)[None, :], mn_ref.shape)
    dmx = jnp.max(jnp.where(valid, d2, -jnp.inf), axis=1)
    tmx = jnp.broadcast_to(dmx[None, :], mx_ref.shape)

    @pl.when(j == 0)
    def _():
        mn_ref[...] = tmn
        mx_ref[...] = tmx

    @pl.when(j > 0)
    def _():
        mn_ref[...] = jnp.minimum(mn_ref[...], tmn)
        mx_ref[...] = jnp.maximum(mx_ref[...], tmx)


def _ladder_kernel(d2_ref, thr_ref, cnt_ref):
    j = pl.program_id(1)
    d2 = d2_ref[...]
    thr = thr_ref[...]
    cols = []
    for jj in range(_J):
        m = (d2 < thr[:, jj:jj + 1]).astype(jnp.float32)
        cols.append(jnp.sum(m, axis=1))
    cnt = jnp.stack(cols, axis=1)

    @pl.when(j == 0)
    def _():
        cnt_ref[...] = cnt

    @pl.when(j > 0)
    def _():
        cnt_ref[...] = cnt_ref[...] + cnt


def _contrib_kernel(d2_ref, thr_ref, lab_ref, w_ref, logp_ref, p_ref, *, nj):
    j = pl.program_id(1)
    d2 = d2_ref[...]
    t = thr_ref[:, 0:1]
    w = w_ref[0, 0, :]
    lab = lab_ref[0, 0, :]
    mask = d2 < t
    vals = jnp.where(mask, jnp.exp(w[None, :] - d2 * GC), 0.0)
    vb = vals.astype(jnp.bfloat16)
    classes = jax.lax.broadcasted_iota(jnp.int32, (lab.shape[0], NUM_CLASSES), 1)
    oh = (lab[:, None] == classes).astype(jnp.bfloat16)
    pt = jax.lax.dot_general(vb, oh, (((1,), (0,)), ((), ())),
                             preferred_element_type=jnp.float32)

    @pl.when(j == 0)
    def _():
        p_ref[...] = pt

    @pl.when(j > 0)
    def _():
        p_ref[...] = p_ref[...] + pt

    @pl.when(j == nj - 1)
    def _():
        p = p_ref[...]
        p = jnp.where(p == 0.0, 1e-10, p)
        p = p / jnp.sum(p, axis=1, keepdims=True)
        p_ref[...] = p
        logp_ref[...] = jnp.log(p)


def _f2b(x):
    return jax.lax.bitcast_convert_type(x, jnp.int32)


def _b2f(x):
    return jax.lax.bitcast_convert_type(x, jnp.float32)


def kernel(features, centres, centre_labels, weight):
    b, d = features.shape
    n = centres.shape[0]
    br = min(_BR, b)
    n_pad = ((n + _BC - 1) // _BC) * _BC
    nj = n_pad // _BC
    grid = (b // br, nj)
    centres_pad = jnp.pad(centres, ((0, n_pad - n), (0, 0)), constant_values=1e3)
    labels_pad = jnp.pad(centre_labels.astype(jnp.int32),
                         (0, n_pad - n)).reshape(nj, 1, _BC)
    weight_pad = jnp.pad(weight, (0, n_pad - n)).reshape(nj, 1, _BC)

    import functools as _ft
    d2, rmn, rmx = pl.pallas_call(
        _ft.partial(_d2_minmax_kernel, n=n),
        grid=grid,
        in_specs=[
            pl.BlockSpec((br, d), lambda i, j: (i, 0)),
            pl.BlockSpec((_BC, d), lambda i, j: (j, 0)),
        ],
        out_specs=[
            pl.BlockSpec((br, _BC), lambda i, j: (i, j)),
            pl.BlockSpec((8, br), lambda i, j: (0, i)),
            pl.BlockSpec((8, br), lambda i, j: (0, i)),
        ],
        out_shape=[
            jax.ShapeDtypeStruct((b, n_pad), jnp.float32),
            jax.ShapeDtypeStruct((8, b), jnp.float32),
            jax.ShapeDtypeStruct((8, b), jnp.float32),
        ],
    )(features, centres_pad)

    lo = _f2b(rmn[0])
    hi = _f2b(rmx[0]) + 1
    knn = jnp.float32(NN)
    found = jnp.zeros((b,), dtype=bool)
    tstar = lo
    cnt_lo = jnp.zeros((b,), jnp.float32)
    cnt_hi = jnp.full((b,), float(n_pad), jnp.float32)

    ladder = pl.pallas_call(
        _ladder_kernel,
        grid=grid,
        in_specs=[
            pl.BlockSpec((br, _BC), lambda i, j: (i, j)),
            pl.BlockSpec((br, _J), lambda i, j: (i, 0)),
        ],
        out_specs=pl.BlockSpec((br, _J), lambda i, j: (i, 0)),
        out_shape=jax.ShapeDtypeStruct((b, _J), jnp.float32),
    )

    steps = jnp.arange(1, _J + 1, dtype=jnp.int32)[None, :]
    for _ in range(_PASSES):
        step = jnp.maximum((hi - lo) // (_J + 1), 1)
        thr_bits = jnp.minimum(lo[:, None] + steps * step[:, None], hi[:, None])
        cnt = ladder(d2, _b2f(thr_bits))
        eq = (cnt == knn) & ~found[:, None]
        anyeq = jnp.any(eq, axis=1)
        idx = jnp.argmax(eq, axis=1)
        t_eq = jnp.take_along_axis(thr_bits, idx[:, None], axis=1)[:, 0]
        tstar = jnp.where(anyeq, t_eq, tstar)
        lt = cnt < knn
        gt = cnt > knn
        lo_new = jnp.max(jnp.where(lt, thr_bits, lo[:, None]), axis=1)
        hi_new = jnp.min(jnp.where(gt, thr_bits, hi[:, None]), axis=1)
        cnt_lo_new = jnp.max(jnp.where(lt, cnt, cnt_lo[:, None]), axis=1)
        cnt_hi_new = jnp.min(jnp.where(gt, cnt, cnt_hi[:, None]), axis=1)
        found = found | anyeq
        lo = jnp.where(found, lo, lo_new)
        hi = jnp.where(found, hi, hi_new)
        cnt_lo = jnp.where(found, cnt_lo, cnt_lo_new)
        cnt_hi = jnp.where(found, cnt_hi, cnt_hi_new)

    tstar = jnp.where(found, tstar,
                      jnp.where(knn - cnt_lo <= cnt_hi - knn, lo, hi))
    thr_pf = jnp.broadcast_to(_b2f(tstar)[:, None], (b, 128))

    logp, p = pl.pallas_call(
        _ft.partial(_contrib_kernel, nj=nj),
        grid=grid,
        in_specs=[
            pl.BlockSpec((br, _BC), lambda i, j: (i, j)),
            pl.BlockSpec((br, 128), lambda i, j: (i, 0)),
            pl.BlockSpec((1, 1, _BC), lambda i, j: (j, 0, 0)),
            pl.BlockSpec((1, 1, _BC), lambda i, j: (j, 0, 0)),
        ],
        out_specs=[
            pl.BlockSpec((br, NUM_CLASSES), lambda i, j: (i, 0)),
            pl.BlockSpec((br, NUM_CLASSES), lambda i, j: (i, 0)),
        ],
        out_shape=[
            jax.ShapeDtypeStruct((b, NUM_CLASSES), jnp.float32),
            jax.ShapeDtypeStruct((b, NUM_CLASSES), jnp.float32),
        ],
    )(d2, thr_pf, labels_pad, weight_pad)
    return (logp, p)


# speedup vs baseline: 14.1693x; 1.0587x over previous
import jax
import jax.numpy as jnp
from jax.experimental import pallas as pl

NUM_CLASSES = 1000
NN = 200
SIGMA = 10.0
GC = 1.0 / (2.0 * SIGMA ** 2)

_BR = 256
_BC = 2048
_J = 16
_PASSES = 5


def _d2_minmax_kernel(q_ref, c_ref, d2_ref, mn_ref, mx_ref, *, n):
    j = pl.program_id(1)
    q = q_ref[...]
    c = c_ref[...]
    q2 = jnp.sum(q * q, axis=1, keepdims=True)
    c2 = jnp.sum(c * c, axis=1)[None, :]
    dot = jax.lax.dot_general(q, c, (((1,), (1,)), ((), ())),
                              preferred_element_type=jnp.float32)
    d2 = q2 + c2 - 2.0 * dot
    d2_ref[...] = d2
    col = j * d2.shape[1] + jax.lax.broadcasted_iota(jnp.int32, d2.shape, 1)
    valid = col < n
    tmn = jnp.broadcast_to(jnp.min(d2, axis=1)[None, :], mn_ref.shape)
    dmx = jnp.max(jnp.where(valid, d2, -jnp.inf), axis=1)
    tmx = jnp.broadcast_to(dmx[None, :], mx_ref.shape)

    @pl.when(j == 0)
    def _():
        mn_ref[...] = tmn
        mx_ref[...] = tmx

    @pl.when(j > 0)
    def _():
        mn_ref[...] = jnp.minimum(mn_ref[...], tmn)
        mx_ref[...] = jnp.maximum(mx_ref[...], tmx)


def _ladder_kernel(d2_ref, thr_ref, cnt_ref):
    j = pl.program_id(1)
    d2 = d2_ref[...]
    thr = thr_ref[...]
    bc = d2.shape[1]
    colid = jax.lax.broadcasted_iota(jnp.int32, (bc, _J), 1)
    cnt = jnp.zeros((d2.shape[0], _J), jnp.float32)
    for jj in range(_J):
        m = (d2 < thr[:, jj:jj + 1]).astype(jnp.float32)
        oj = (colid == jj).astype(jnp.float32)
        cnt = cnt + jax.lax.dot_general(m, oj, (((1,), (0,)), ((), ())),
                                        preferred_element_type=jnp.float32)

    @pl.when(j == 0)
    def _():
        cnt_ref[...] = cnt

    @pl.when(j > 0)
    def _():
        cnt_ref[...] = cnt_ref[...] + cnt


def _contrib_kernel(d2_ref, thr_ref, lab_ref, w_ref, logp_ref, p_ref, *, nj):
    j = pl.program_id(1)
    d2 = d2_ref[...]
    t = thr_ref[:, 0:1]
    w = w_ref[0, 0, :]
    lab = lab_ref[0, 0, :]
    mask = d2 < t
    vals = jnp.where(mask, jnp.exp(w[None, :] - d2 * GC), 0.0)
    vb = vals.astype(jnp.bfloat16)
    classes = jax.lax.broadcasted_iota(jnp.int32, (lab.shape[0], NUM_CLASSES), 1)
    oh = (lab[:, None] == classes).astype(jnp.bfloat16)
    pt = jax.lax.dot_general(vb, oh, (((1,), (0,)), ((), ())),
                             preferred_element_type=jnp.float32)

    @pl.when(j == 0)
    def _():
        p_ref[...] = pt

    @pl.when(j > 0)
    def _():
        p_ref[...] = p_ref[...] + pt

    @pl.when(j == nj - 1)
    def _():
        p = p_ref[...]
        p = jnp.where(p == 0.0, 1e-10, p)
        p = p / jnp.sum(p, axis=1, keepdims=True)
        p_ref[...] = p
        logp_ref[...] = jnp.log(p)


def _f2b(x):
    return jax.lax.bitcast_convert_type(x, jnp.int32)


def _b2f(x):
    return jax.lax.bitcast_convert_type(x, jnp.float32)


def kernel(features, centres, centre_labels, weight):
    b, d = features.shape
    n = centres.shape[0]
    br = min(_BR, b)
    n_pad = ((n + _BC - 1) // _BC) * _BC
    nj = n_pad // _BC
    grid = (b // br, nj)
    centres_pad = jnp.pad(centres, ((0, n_pad - n), (0, 0)), constant_values=1e3)
    labels_pad = jnp.pad(centre_labels.astype(jnp.int32),
                         (0, n_pad - n)).reshape(nj, 1, _BC)
    weight_pad = jnp.pad(weight, (0, n_pad - n)).reshape(nj, 1, _BC)

    import functools as _ft
    d2, rmn, rmx = pl.pallas_call(
        _ft.partial(_d2_minmax_kernel, n=n),
        grid=grid,
        in_specs=[
            pl.BlockSpec((br, d), lambda i, j: (i, 0)),
            pl.BlockSpec((_BC, d), lambda i, j: (j, 0)),
        ],
        out_specs=[
            pl.BlockSpec((br, _BC), lambda i, j: (i, j)),
            pl.BlockSpec((8, br), lambda i, j: (0, i)),
            pl.BlockSpec((8, br), lambda i, j: (0, i)),
        ],
        out_shape=[
            jax.ShapeDtypeStruct((b, n_pad), jnp.float32),
            jax.ShapeDtypeStruct((8, b), jnp.float32),
            jax.ShapeDtypeStruct((8, b), jnp.float32),
        ],
    )(features, centres_pad)

    lo = _f2b(rmn[0])
    hi = _f2b(rmx[0]) + 1
    knn = jnp.float32(NN)
    found = jnp.zeros((b,), dtype=bool)
    tstar = lo
    cnt_lo = jnp.zeros((b,), jnp.float32)
    cnt_hi = jnp.full((b,), float(n_pad), jnp.float32)

    ladder = pl.pallas_call(
        _ladder_kernel,
        grid=grid,
        in_specs=[
            pl.BlockSpec((br, _BC), lambda i, j: (i, j)),
            pl.BlockSpec((br, _J), lambda i, j: (i, 0)),
        ],
        out_specs=pl.BlockSpec((br, _J), lambda i, j: (i, 0)),
        out_shape=jax.ShapeDtypeStruct((b, _J), jnp.float32),
    )

    steps = jnp.arange(1, _J + 1, dtype=jnp.int32)[None, :]
    for _ in range(_PASSES):
        step = jnp.maximum((hi - lo) // (_J + 1), 1)
        thr_bits = jnp.minimum(lo[:, None] + steps * step[:, None], hi[:, None])
        cnt = ladder(d2, _b2f(thr_bits))
        eq = (cnt == knn) & ~found[:, None]
        anyeq = jnp.any(eq, axis=1)
        idx = jnp.argmax(eq, axis=1)
        t_eq = jnp.take_along_axis(thr_bits, idx[:, None], axis=1)[:, 0]
        tstar = jnp.where(anyeq, t_eq, tstar)
        lt = cnt < knn
        gt = cnt > knn
        lo_new = jnp.max(jnp.where(lt, thr_bits, lo[:, None]), axis=1)
        hi_new = jnp.min(jnp.where(gt, thr_bits, hi[:, None]), axis=1)
        cnt_lo_new = jnp.max(jnp.where(lt, cnt, cnt_lo[:, None]), axis=1)
        cnt_hi_new = jnp.min(jnp.where(gt, cnt, cnt_hi[:, None]), axis=1)
        found = found | anyeq
        lo = jnp.where(found, lo, lo_new)
        hi = jnp.where(found, hi, hi_new)
        cnt_lo = jnp.where(found, cnt_lo, cnt_lo_new)
        cnt_hi = jnp.where(found, cnt_hi, cnt_hi_new)

    tstar = jnp.where(found, tstar,
                      jnp.where(knn - cnt_lo <= cnt_hi - knn, lo, hi))
    thr_pf = jnp.broadcast_to(_b2f(tstar)[:, None], (b, 128))

    logp, p = pl.pallas_call(
        _ft.partial(_contrib_kernel, nj=nj),
        grid=grid,
        in_specs=[
            pl.BlockSpec((br, _BC), lambda i, j: (i, j)),
            pl.BlockSpec((br, 128), lambda i, j: (i, 0)),
            pl.BlockSpec((1, 1, _BC), lambda i, j: (j, 0, 0)),
            pl.BlockSpec((1, 1, _BC), lambda i, j: (j, 0, 0)),
        ],
        out_specs=[
            pl.BlockSpec((br, NUM_CLASSES), lambda i, j: (i, 0)),
            pl.BlockSpec((br, NUM_CLASSES), lambda i, j: (i, 0)),
        ],
        out_shape=[
            jax.ShapeDtypeStruct((b, NUM_CLASSES), jnp.float32),
            jax.ShapeDtypeStruct((b, NUM_CLASSES), jnp.float32),
        ],
    )(d2, thr_pf, labels_pad, weight_pad)
    return (logp, p)
